# Initial kernel scaffold; baseline (speedup 1.0000x reference)
import jax
import jax.numpy as jnp
from jax.experimental import pallas as pl


def kernel(x, edge_index, W_gcn, b_gcn, W_sage_l, b_sage_l, W_sage_r, W_gin1, b_gin1, W_gin2, b_gin2, W_gat, a_src, a_dst, b_gat):
    raise NotImplementedError("write your pallas kernel here")


# speedup vs baseline: 8.0092x; 8.0092x over previous
import functools

import jax
import jax.numpy as jnp
from jax import lax
from jax.experimental import pallas as pl
from jax.experimental.pallas import tpu as pltpu
from jax.experimental.pallas import tpu_sc as plsc

N = 10000
E = 320000
D = 128
H = D // 2
NC = 2
NS = 16
NW = NC * NS
SUB = 128
EPT2 = 10240
E2 = EPT2 * NW
SUB2 = EPT2 // SUB
EPT4 = E2 // NS
SUB4 = EPT4 // SUB
R = 10240
RPT = R // NS
BLK = 2000

_f32 = jnp.float32
_i32 = jnp.int32


def _k1_body(x_ref, wg_ref, a2_ref, out_ref, m_ref):
    i = pl.program_id(0)
    wa = jnp.dot(wg_ref[...], a2_ref[...], preferred_element_type=_f32)
    al = jnp.dot(x_ref[...], wa, preferred_element_type=_f32)
    out_ref[...] = al

    @pl.when(i == 0)
    def _():
        m_ref[...] = jnp.full((2, 128), -3e38, _f32)
    ms = jnp.full((1, 128), jnp.max(al[:, 0:1]), _f32)
    md = jnp.full((1, 128), jnp.max(al[:, 1:2]), _f32)
    m_ref[...] = jnp.maximum(m_ref[...], jnp.concatenate([ms, md], axis=0))


def _k1(x, W_gat, a2):
    return pl.pallas_call(
        _k1_body,
        grid=(N // BLK,),
        in_specs=[
            pl.BlockSpec((BLK, D), lambda i: (i, 0)),
            pl.BlockSpec((D, D), lambda i: (0, 0)),
            pl.BlockSpec((D, 2), lambda i: (0, 0)),
        ],
        out_specs=[pl.BlockSpec((BLK, 2), lambda i: (i, 0)),
                   pl.BlockSpec((2, 128), lambda i: (0, 0))],
        out_shape=[jax.ShapeDtypeStruct((N, 2), _f32),
                   jax.ShapeDtypeStruct((2, 128), _f32)],
    )(x, W_gat, a2)


def _k2_body(src_hbm, dst_hbm, als_hbm, ald_hbm, m_hbm,
             ee_hbm, acc_hbm, ees_hbm,
             als_v, ald_v, srcb, dstb, eeb, valb, mbuf, eesb, accb, packb,
             idxb, sem0, acc_sh):
    c = lax.axis_index("c")
    s = lax.axis_index("s")
    w = c * NS + s

    zero16 = jnp.zeros((16,), _f32)
    iota16 = lax.iota(_i32, 16)

    pltpu.sync_copy(src_hbm.at[w], srcb)
    pltpu.sync_copy(dst_hbm.at[w], dstb)
    pltpu.sync_copy(als_hbm, als_v.at[pl.ds(0, N)])
    pltpu.sync_copy(ald_hbm, ald_v.at[pl.ds(0, N)])
    pltpu.sync_copy(m_hbm, mbuf)

    def _ztail(i, _):
        als_v[pl.ds(N + i * 16, 16)] = zero16
        ald_v[pl.ds(N + i * 16, 16)] = zero16
        return 0
    lax.fori_loop(0, (R - N) // 16, _ztail, 0)

    tm = mbuf[0, pl.ds(0, 16)] + mbuf[1, pl.ds(0, 16)]
    M = jnp.maximum(tm, 0.2 * tm)

    def _zval(i, _):
        valb[i] = zero16
        return 0
    lax.fori_loop(0, SUB, _zval, 0)
    for g in range(RPT // SUB):
        base = s * RPT + g * SUB
        for k in range(SUB // 16):
            idxb[pl.ds(k * 16, 16)] = iota16 + (base + k * 16)
        pltpu.sync_copy(valb, acc_sh.at[idxb])
    plsc.subcore_barrier()

    mask0 = iota16 == 0

    def _chunk(j, _):
        for k in range(SUB // 16):
            sv = srcb[j, pl.ds(k * 16, 16)]
            dv = dstb[j, pl.ds(k * 16, 16)]
            a = plsc.load_gather(als_v, [sv])
            b = plsc.load_gather(ald_v, [dv])
            t = a + b
            ee = jnp.exp(jnp.maximum(t, 0.2 * t) - M)
            eeb[j, pl.ds(k * 16, 16)] = ee
            for l in range(16):
                r = jnp.full((16,), ee[l], _f32)
                valb[k * 16 + l] = jnp.where(mask0, 1.0, r)
        pltpu.sync_copy(valb, acc_sh.at[dstb.at[j]], add=True)
        return 0
    lax.fori_loop(0, SUB2, _chunk, 0)

    pltpu.sync_copy(eeb, ee_hbm.at[w])

    @pl.when(c == 0)
    def _():
        def _ees(k, _):
            a = als_v[pl.ds(s * RPT + k * 16, 16)]
            b = ald_v[pl.ds(s * RPT + k * 16, 16)]
            t = a + b
            eesb[pl.ds(k * 16, 16)] = jnp.exp(jnp.maximum(t, 0.2 * t) - M)
            return 0
        lax.fori_loop(0, RPT // 16, _ees, 0)
        pltpu.sync_copy(eesb, ees_hbm.at[s])

    plsc.subcore_barrier()
    for g in range(RPT // SUB):
        base = s * RPT + g * SUB
        for k in range(SUB // 16):
            idxb[pl.ds(k * 16, 16)] = iota16 + (base + k * 16)
        pltpu.async_copy(acc_sh.at[idxb], accb, sem0).wait()
        def _packa(i, _):
            for k in range(8):
                packb[i, pl.ds(k * 16, 16)] = accb[8 * i + k]
            return 0
        lax.fori_loop(0, SUB // 8, _packa, 0)
        pltpu.sync_copy(packb, acc_hbm.at[c, s, pl.ds(g * (SUB // 8), SUB // 8)])


def _k2(src2, dst2, als, ald, m2):
    mesh = plsc.VectorSubcoreMesh(core_axis_name="c", subcore_axis_name="s",
                                  num_cores=NC, num_subcores=NS)
    f = pl.kernel(
        _k2_body,
        out_type=(jax.ShapeDtypeStruct((NW, SUB2, SUB), _f32),
                  jax.ShapeDtypeStruct((NC, NS, RPT // 8, 128), _f32),
                  jax.ShapeDtypeStruct((NS, RPT), _f32)),
        mesh=mesh,
        compiler_params=pltpu.CompilerParams(needs_layout_passes=False),
        scratch_types=[
            pltpu.VMEM((R,), _f32),
            pltpu.VMEM((R,), _f32),
            pltpu.VMEM((SUB2, SUB), _i32),
            pltpu.VMEM((SUB2, SUB), _i32),
            pltpu.VMEM((SUB2, SUB), _f32),
            pltpu.VMEM((SUB, 16), _f32),
            pltpu.VMEM((2, 128), _f32),
            pltpu.VMEM((RPT,), _f32),
            pltpu.VMEM((SUB, 16), _f32),
            pltpu.VMEM((16, 128), _f32),
            pltpu.VMEM((SUB,), _i32),
            pltpu.SemaphoreType.DMA,
            pltpu.VMEM_SHARED((R, 16), _f32),
        ],
    )
    return f(src2, dst2, als, ald, m2)


def _k3_body(x_ref, acc_ref, ees_ref, z_ref, np_ref):
    a0 = acc_ref[0]
    a1 = acc_ref[1]
    ind = a0[:, 0:1] + a1[:, 0:1]
    dene = a0[:, 1:2] + a1[:, 1:2]
    ees = ees_ref[...]
    dinv = lax.rsqrt(ind + 1.0)
    sage = 1.0 / jnp.maximum(ind, 1.0)
    invden = 1.0 / (dene + ees)
    z_ref[...] = x_ref[...] * dinv
    np_ref[...] = jnp.concatenate(
        [dinv, sage, invden, ees, jnp.zeros_like(ind), jnp.zeros_like(ind),
         jnp.zeros_like(ind), jnp.zeros_like(ind)], axis=1)


def _k3(x, acc, ees_col):
    return pl.pallas_call(
        _k3_body,
        grid=(N // BLK,),
        in_specs=[
            pl.BlockSpec((BLK, D), lambda i: (i, 0)),
            pl.BlockSpec((2, BLK, 16), lambda i: (0, i, 0)),
            pl.BlockSpec((BLK, 1), lambda i: (i, 0)),
        ],
        out_specs=[
            pl.BlockSpec((BLK, D), lambda i: (i, 0)),
            pl.BlockSpec((BLK, 8), lambda i: (i, 0)),
        ],
        out_shape=[jax.ShapeDtypeStruct((N, D), _f32),
                   jax.ShapeDtypeStruct((N, 8), _f32)],
    )(x, acc, ees_col)


NPP = R // 2
R2 = 5376
ZPT = R2 // NS
RPT2 = NPP // NS
GRP = 16


def _k4_body(src_hbm, dst_hbm, ee_hbm, xz_hbm,
             st_hbm, u_hbm,
             srcb, dstb, eeb, gxz, v3, dstb2, ub, packb, idxz, idxo, sem0,
             st_sh, u_sh):
    c = lax.axis_index("c")
    s = lax.axis_index("s")

    zero16 = jnp.zeros((16,), _f32)
    iota16 = lax.iota(_i32, 16)

    for p in range(2):
        def _zg(i, _):
            for k in range(8):
                gxz[i, pl.ds(k * 16, 16)] = zero16
            for k in range(4):
                v3[i, pl.ds(k * 16, 16)] = zero16
            return 0
        lax.fori_loop(0, SUB, _zg, 0)
        for g in range(3):
            base = s * ZPT + g * 112
            for k in range(7):
                idxz[pl.ds(k * 16, 16)] = iota16 + (base + k * 16)
            pltpu.sync_copy(gxz.at[pl.ds(0, 112)], st_sh.at[idxz])
            pltpu.sync_copy(v3.at[pl.ds(0, 112)], u_sh.at[idxz])
        plsc.subcore_barrier()

        def _group(g, _):
            pltpu.sync_copy(src_hbm.at[c, s, pl.ds(g * GRP, GRP)], srcb)
            pltpu.sync_copy(dst_hbm.at[s, pl.ds(g * GRP, GRP)], dstb)
            pltpu.sync_copy(ee_hbm.at[s, pl.ds(g * GRP, GRP)], eeb)

            def _chunk(j, _):
                pltpu.async_copy(xz_hbm.at[srcb.at[j]], gxz, sem0).wait()
                for k in range(SUB // 16):
                    d = dstb[j, pl.ds(k * 16, 16)] - (p * NPP)
                    ok = (d >= 0) & (d < NPP)
                    dstb2[pl.ds(k * 16, 16)] = jnp.where(ok, d, NPP)
                def _we(q, _):
                    wvec = eeb[j, pl.ds(q * 16, 16)]
                    for l in range(16):
                        e = q * 16 + l
                        wv = jnp.full((16,), wvec[l], _f32)
                        for k in range(4):
                            v3[e, pl.ds(k * 16, 16)] = (
                                gxz[e, pl.ds(k * 16, 16)] * wv)
                    return 0
                lax.fori_loop(0, SUB // 16, _we, 0)
                pltpu.sync_copy(gxz, st_sh.at[dstb2], add=True)
                pltpu.sync_copy(v3, u_sh.at[dstb2], add=True)
                return 0
            lax.fori_loop(0, GRP, _chunk, 0)
            return 0
        lax.fori_loop(0, SUB4 // GRP, _group, 0)

        plsc.subcore_barrier()
        for g in range(5):
            base = s * RPT2 + g * 64
            for k in range(4):
                idxo[pl.ds(k * 16, 16)] = iota16 + (base + k * 16)
            pltpu.async_copy(st_sh.at[idxo], gxz.at[pl.ds(0, 64)], sem0).wait()
            pltpu.sync_copy(gxz.at[pl.ds(0, 64)],
                            st_hbm.at[c, p, pl.ds(base, 64)])
            pltpu.async_copy(u_sh.at[idxo], ub.at[pl.ds(0, 64)], sem0).wait()
            def _pack(i, _):
                for k in range(8):
                    packb[i, pl.ds(k * 16, 16)] = ub[2 * i + k // 4,
                                                     pl.ds((k % 4) * 16, 16)]
                return 0
            lax.fori_loop(0, 32, _pack, 0)
            pltpu.sync_copy(packb.at[pl.ds(0, 32)],
                            u_hbm.at[c, p, s, pl.ds(g * 32, 32)])
        plsc.subcore_barrier()


def _k4(src4b, dst4, ee4, XZcat):
    mesh = plsc.VectorSubcoreMesh(core_axis_name="c", subcore_axis_name="s",
                                  num_cores=NC, num_subcores=NS)
    f = pl.kernel(
        _k4_body,
        out_type=(jax.ShapeDtypeStruct((NC, 2, NPP, 128), _f32),
                  jax.ShapeDtypeStruct((NC, 2, NS, RPT2 // 2, 128), _f32)),
        mesh=mesh,
        compiler_params=pltpu.CompilerParams(needs_layout_passes=False),
        scratch_types=[
            pltpu.VMEM((GRP, SUB), _i32),
            pltpu.VMEM((GRP, SUB), _i32),
            pltpu.VMEM((GRP, SUB), _f32),
            pltpu.VMEM((SUB, 128), _f32),
            pltpu.VMEM((SUB, H), _f32),
            pltpu.VMEM((SUB,), _i32),
            pltpu.VMEM((SUB, H), _f32),
            pltpu.VMEM((SUB // 2, 128), _f32),
            pltpu.VMEM((112,), _i32),
            pltpu.VMEM((64,), _i32),
            pltpu.SemaphoreType.DMA,
            pltpu.VMEM_SHARED((R2, 128), _f32),
            pltpu.VMEM_SHARED((R2, H), _f32),
        ],
    )
    return f(src4b, dst4, ee4, XZcat)


def _k5_body(x_ref, s_ref, t_ref, u_ref, np_ref,
             wgcn, bgcn, wsl, bsl, wsr, wg1, bg1, wg2, bg2, wgat, bgat,
             out_ref):
    x = x_ref[...]
    S = s_ref[...]
    T = t_ref[...]
    U = u_ref[...]
    npb = np_ref[...]
    dinv = npb[:, 0:1]
    sage = npb[:, 1:2]
    invden = npb[:, 2:3]
    ees = npb[:, 3:4]

    dot = functools.partial(jnp.dot, preferred_element_type=_f32)
    x1 = dot(dinv * T + (dinv * dinv) * x, wgcn[...]) + bgcn[...]
    x2 = dot(sage * S, wsl[...]) + bsl[...] + dot(x, wsr[...])
    hg = jnp.maximum(dot(x + S, wg1[...]) + bg1[...], 0.0)
    x3 = dot(hg, wg2[...]) + bg2[...]
    x4 = dot((U + ees * x) * invden, wgat[...]) + bgat[...]
    out_ref[...] = jnp.maximum(x1 + x2 + x3 + x4, 0.0)


def _k5(x, S, T, U, nodep, W_gcn, b_gcn, W_sage_l, b_sage_l, W_sage_r,
        W_gin1, b_gin1, W_gin2, b_gin2, W_gat, b_gat):
    full = lambda shape: pl.BlockSpec(shape, lambda i: tuple(0 for _ in shape))
    row = pl.BlockSpec((BLK, D), lambda i: (i, 0))
    return pl.pallas_call(
        _k5_body,
        grid=(N // BLK,),
        in_specs=[
            row, row, row, row,
            pl.BlockSpec((BLK, 8), lambda i: (i, 0)),
            full((D, D)), full((1, D)),
            full((D, D)), full((1, D)), full((D, D)),
            full((D, D)), full((1, D)), full((D, D)), full((1, D)),
            full((D, D)), full((1, D)),
        ],
        out_specs=row,
        out_shape=jax.ShapeDtypeStruct((N, D), _f32),
    )(x, S, T, U, nodep,
      W_gcn, b_gcn, W_sage_l, b_sage_l, W_sage_r,
      W_gin1, b_gin1, W_gin2, b_gin2, W_gat, b_gat)


def kernel(x, edge_index, W_gcn, b_gcn, W_sage_l, b_sage_l, W_sage_r,
           W_gin1, b_gin1, W_gin2, b_gin2, W_gat, a_src, a_dst, b_gat):
    src = edge_index[0]
    dst = edge_index[1]
    pad = E2 - E
    src_p = jnp.concatenate([src, jnp.zeros((pad,), _i32)])
    dst_p = jnp.concatenate([dst, jnp.full((pad,), N, _i32)])

    a2 = jnp.stack([a_src, a_dst], axis=1)
    al2, m2 = _k1(x, W_gat, a2)
    als = al2[:, 0] + 0.0
    ald = al2[:, 1] + 0.0

    src2 = src_p.reshape(NW, SUB2, SUB)
    dst2 = dst_p.reshape(NW, SUB2, SUB)
    ee, acc4, eeself = _k2(src2, dst2, als, ald, m2)

    acc = acc4.reshape(NC, R, 16)
    ees_col = eeself.reshape(R, 1)[:N]
    z, nodep = _k3(x, acc, ees_col)

    XZcat = jnp.concatenate(
        [jnp.concatenate([x[:, :H], z[:, :H]], axis=1),
         jnp.concatenate([x[:, H:], z[:, H:]], axis=1)], axis=0)
    src4 = src_p.reshape(NS, SUB4, SUB)
    src4b = jnp.stack([src4, src4 + N], axis=0)
    dst4 = dst_p.reshape(NS, SUB4, SUB)
    ee4 = ee.reshape(NS, SUB4, SUB)

    st4, u4 = _k4(src4b, dst4, ee4, XZcat)
    st = st4.reshape(NC, R, 128)
    S = jnp.concatenate([st[0, :N, :H], st[1, :N, :H]], axis=1)
    T = jnp.concatenate([st[0, :N, H:], st[1, :N, H:]], axis=1)
    u_r = u4.reshape(NC, R, H)
    U = jnp.concatenate([u_r[0, :N], u_r[1, :N]], axis=1)

    r2 = lambda b: b.reshape(1, D)
    return _k5(x, S, T, U, nodep, W_gcn, r2(b_gcn), W_sage_l, r2(b_sage_l),
               W_sage_r, W_gin1, r2(b_gin1), W_gin2, r2(b_gin2), W_gat,
               r2(b_gat))


# speedup vs baseline: 8.8094x; 1.0999x over previous
import functools

import jax
import jax.numpy as jnp
from jax import lax
from jax.experimental import pallas as pl
from jax.experimental.pallas import tpu as pltpu
from jax.experimental.pallas import tpu_sc as plsc

N = 10000
E = 320000
D = 128
H = D // 2
NC = 2
NS = 16
NW = NC * NS
SUB = 128
EPT2 = 10240
E2 = EPT2 * NW
SUB2 = EPT2 // SUB
EPT4 = E2 // NS
SUB4 = EPT4 // SUB
R = 10240
RPT = R // NS
BLK = 2000

_f32 = jnp.float32
_i32 = jnp.int32


def _k1_body(x_ref, wg_ref, a2_ref, out_ref, m_ref):
    i = pl.program_id(0)
    wa = jnp.dot(wg_ref[...], a2_ref[...], preferred_element_type=_f32)
    al = jnp.dot(x_ref[...], wa, preferred_element_type=_f32)
    out_ref[...] = al

    @pl.when(i == 0)
    def _():
        m_ref[...] = jnp.full((2, 128), -3e38, _f32)
    ms = jnp.full((1, 128), jnp.max(al[:, 0:1]), _f32)
    md = jnp.full((1, 128), jnp.max(al[:, 1:2]), _f32)
    m_ref[...] = jnp.maximum(m_ref[...], jnp.concatenate([ms, md], axis=0))


def _k1(x, W_gat, a2):
    return pl.pallas_call(
        _k1_body,
        grid=(N // BLK,),
        in_specs=[
            pl.BlockSpec((BLK, D), lambda i: (i, 0)),
            pl.BlockSpec((D, D), lambda i: (0, 0)),
            pl.BlockSpec((D, 2), lambda i: (0, 0)),
        ],
        out_specs=[pl.BlockSpec((BLK, 2), lambda i: (i, 0)),
                   pl.BlockSpec((2, 128), lambda i: (0, 0))],
        out_shape=[jax.ShapeDtypeStruct((N, 2), _f32),
                   jax.ShapeDtypeStruct((2, 128), _f32)],
    )(x, W_gat, a2)


def _k2_body(src_hbm, dst_hbm, als_hbm, ald_hbm, m_hbm,
             ee_hbm, acc_hbm, ees_hbm,
             als_v, ald_v, srcb, dstb, eeb, valb, mbuf, eesb, accb, packb,
             idxb, sem0, acc_sh):
    c = lax.axis_index("c")
    s = lax.axis_index("s")
    w = c * NS + s

    zero16 = jnp.zeros((16,), _f32)
    iota16 = lax.iota(_i32, 16)

    pltpu.sync_copy(src_hbm.at[w], srcb)
    pltpu.sync_copy(dst_hbm.at[w], dstb)
    pltpu.sync_copy(als_hbm, als_v.at[pl.ds(0, N)])
    pltpu.sync_copy(ald_hbm, ald_v.at[pl.ds(0, N)])
    pltpu.sync_copy(m_hbm, mbuf)

    def _ztail(i, _):
        als_v[pl.ds(N + i * 16, 16)] = zero16
        ald_v[pl.ds(N + i * 16, 16)] = zero16
        return 0
    lax.fori_loop(0, (R - N) // 16, _ztail, 0)

    tm = mbuf[0, pl.ds(0, 16)] + mbuf[1, pl.ds(0, 16)]
    M = jnp.maximum(tm, 0.2 * tm)

    def _zval(i, _):
        valb[i] = zero16
        return 0
    lax.fori_loop(0, SUB, _zval, 0)
    for g in range(RPT // SUB):
        base = s * RPT + g * SUB
        for k in range(SUB // 16):
            idxb[pl.ds(k * 16, 16)] = iota16 + (base + k * 16)
        pltpu.sync_copy(valb, acc_sh.at[idxb])
    plsc.subcore_barrier()

    mask0 = iota16 == 0

    def _chunk(j, _):
        for k in range(SUB // 16):
            sv = srcb[j, pl.ds(k * 16, 16)]
            dv = dstb[j, pl.ds(k * 16, 16)]
            a = plsc.load_gather(als_v, [sv])
            b = plsc.load_gather(ald_v, [dv])
            t = a + b
            ee = jnp.exp(jnp.maximum(t, 0.2 * t) - M)
            eeb[j, pl.ds(k * 16, 16)] = ee
            for l in range(16):
                r = jnp.full((16,), ee[l], _f32)
                valb[k * 16 + l] = jnp.where(mask0, 1.0, r)
        pltpu.sync_copy(valb, acc_sh.at[dstb.at[j]], add=True)
        return 0
    lax.fori_loop(0, SUB2, _chunk, 0)

    pltpu.sync_copy(eeb, ee_hbm.at[w])

    @pl.when(c == 0)
    def _():
        def _ees(k, _):
            a = als_v[pl.ds(s * RPT + k * 16, 16)]
            b = ald_v[pl.ds(s * RPT + k * 16, 16)]
            t = a + b
            eesb[pl.ds(k * 16, 16)] = jnp.exp(jnp.maximum(t, 0.2 * t) - M)
            return 0
        lax.fori_loop(0, RPT // 16, _ees, 0)
        pltpu.sync_copy(eesb, ees_hbm.at[s])

    plsc.subcore_barrier()
    for g in range(RPT // SUB):
        base = s * RPT + g * SUB
        for k in range(SUB // 16):
            idxb[pl.ds(k * 16, 16)] = iota16 + (base + k * 16)
        pltpu.async_copy(acc_sh.at[idxb], accb, sem0).wait()
        def _packa(i, _):
            for k in range(8):
                packb[i, pl.ds(k * 16, 16)] = accb[8 * i + k]
            return 0
        lax.fori_loop(0, SUB // 8, _packa, 0)
        pltpu.sync_copy(packb, acc_hbm.at[c, s, pl.ds(g * (SUB // 8), SUB // 8)])


def _k2(src2, dst2, als, ald, m2):
    mesh = plsc.VectorSubcoreMesh(core_axis_name="c", subcore_axis_name="s",
                                  num_cores=NC, num_subcores=NS)
    f = pl.kernel(
        _k2_body,
        out_type=(jax.ShapeDtypeStruct((NW, SUB2, SUB), _f32),
                  jax.ShapeDtypeStruct((NC, NS, RPT // 8, 128), _f32),
                  jax.ShapeDtypeStruct((NS, RPT), _f32)),
        mesh=mesh,
        compiler_params=pltpu.CompilerParams(needs_layout_passes=False),
        scratch_types=[
            pltpu.VMEM((R,), _f32),
            pltpu.VMEM((R,), _f32),
            pltpu.VMEM((SUB2, SUB), _i32),
            pltpu.VMEM((SUB2, SUB), _i32),
            pltpu.VMEM((SUB2, SUB), _f32),
            pltpu.VMEM((SUB, 16), _f32),
            pltpu.VMEM((2, 128), _f32),
            pltpu.VMEM((RPT,), _f32),
            pltpu.VMEM((SUB, 16), _f32),
            pltpu.VMEM((16, 128), _f32),
            pltpu.VMEM((SUB,), _i32),
            pltpu.SemaphoreType.DMA,
            pltpu.VMEM_SHARED((R, 16), _f32),
        ],
    )
    return f(src2, dst2, als, ald, m2)


def _k3_body(x_ref, acc_ref, ees_ref, z_ref, np_ref):
    a0 = acc_ref[0]
    a1 = acc_ref[1]
    ind = a0[:, 0:1] + a1[:, 0:1]
    dene = a0[:, 1:2] + a1[:, 1:2]
    ees = ees_ref[...]
    dinv = lax.rsqrt(ind + 1.0)
    sage = 1.0 / jnp.maximum(ind, 1.0)
    invden = 1.0 / (dene + ees)
    z_ref[...] = x_ref[...] * dinv
    np_ref[...] = jnp.concatenate(
        [dinv, sage, invden, ees, jnp.zeros_like(ind), jnp.zeros_like(ind),
         jnp.zeros_like(ind), jnp.zeros_like(ind)], axis=1)


def _k3(x, acc, ees_col):
    return pl.pallas_call(
        _k3_body,
        grid=(N // BLK,),
        in_specs=[
            pl.BlockSpec((BLK, D), lambda i: (i, 0)),
            pl.BlockSpec((2, BLK, 16), lambda i: (0, i, 0)),
            pl.BlockSpec((BLK, 1), lambda i: (i, 0)),
        ],
        out_specs=[
            pl.BlockSpec((BLK, D), lambda i: (i, 0)),
            pl.BlockSpec((BLK, 8), lambda i: (i, 0)),
        ],
        out_shape=[jax.ShapeDtypeStruct((N, D), _f32),
                   jax.ShapeDtypeStruct((N, 8), _f32)],
    )(x, acc, ees_col)


NPP = R // 2
R2 = 5376
ZPT = R2 // NS
RPT2 = NPP // NS
GRP = 16


def _k4_body(src_hbm, dst_hbm, ee_hbm, xz_hbm,
             st_hbm, u_hbm,
             srcb, dstb, eeb, gxz0, gxz1, v3, d2a, d2b, idxz, idxo,
             gs0, gs1, ss0, ss1, us0, us1,
             st_sh, u_sh):
    c = lax.axis_index("c")
    s = lax.axis_index("s")

    zero16 = jnp.zeros((16,), _f32)
    iota16 = lax.iota(_i32, 16)

    def _adj(j, d2ref, p):
        for k in range(SUB // 16):
            d = dstb[j, pl.ds(k * 16, 16)] - (p * NPP)
            ok = (d >= 0) & (d < NPP)
            d2ref[pl.ds(k * 16, 16)] = jnp.where(ok, d, NPP)

    def _wei(j, gref, vref):
        def _we(q, _):
            wvec = eeb[j, pl.ds(q * 16, 16)]
            for l in range(16):
                e = q * 16 + l
                wv = jnp.full((16,), wvec[l], _f32)
                for k in range(4):
                    vref[e, pl.ds(k * 16, 16)] = (
                        gref[e, pl.ds(k * 16, 16)] * wv)
            return 0
        lax.fori_loop(0, SUB // 16, _we, 0)

    for p in range(2):
        def _zg(i, _):
            for k in range(8):
                gxz0[i, pl.ds(k * 16, 16)] = zero16
            for k in range(4):
                v3[i, pl.ds(k * 16, 16)] = zero16
            return 0
        lax.fori_loop(0, SUB, _zg, 0)
        for g in range(3):
            base = s * ZPT + g * 112
            for k in range(7):
                idxz[pl.ds(k * 16, 16)] = iota16 + (base + k * 16)
            pltpu.sync_copy(gxz0.at[pl.ds(0, 112)], st_sh.at[idxz])
            pltpu.sync_copy(v3.at[pl.ds(0, 112)], u_sh.at[idxz])
        plsc.subcore_barrier()

        def _group(g, _):
            pltpu.sync_copy(src_hbm.at[c, s, pl.ds(g * GRP, GRP)], srcb)
            pltpu.sync_copy(dst_hbm.at[s, pl.ds(g * GRP, GRP)], dstb)
            pltpu.sync_copy(ee_hbm.at[s, pl.ds(g * GRP, GRP)], eeb)
            pltpu.async_copy(xz_hbm.at[srcb.at[0]], gxz0, gs0)

            def _pair(jj, _):
                a = 2 * jj
                b = 2 * jj + 1
                pltpu.make_async_copy(xz_hbm.at[srcb.at[a]], gxz0, gs0).wait()
                pltpu.async_copy(xz_hbm.at[srcb.at[b]], gxz1, gs1)
                _adj(a, d2a, p)
                _wei(a, gxz0, v3)
                pltpu.async_copy(gxz0, st_sh.at[d2a], ss0, add=True)
                pltpu.async_copy(v3, u_sh.at[d2a], us0, add=True)
                pltpu.make_async_copy(xz_hbm.at[srcb.at[b]], gxz1, gs1).wait()
                _adj(b, d2b, p)
                pltpu.make_async_copy(v3, u_sh.at[d2a], us0).wait()
                _wei(b, gxz1, v3)
                pltpu.make_async_copy(gxz0, st_sh.at[d2a], ss0).wait()

                @pl.when(jj < GRP // 2 - 1)
                def _():
                    pltpu.async_copy(xz_hbm.at[srcb.at[a + 2]], gxz0, gs0)
                pltpu.async_copy(gxz1, st_sh.at[d2b], ss1, add=True)
                pltpu.async_copy(v3, u_sh.at[d2b], us1, add=True)
                pltpu.make_async_copy(gxz1, st_sh.at[d2b], ss1).wait()
                pltpu.make_async_copy(v3, u_sh.at[d2b], us1).wait()
                return 0
            lax.fori_loop(0, GRP // 2, _pair, 0)
            return 0
        lax.fori_loop(0, SUB4 // GRP, _group, 0)

        plsc.subcore_barrier()
        for g in range(5):
            base = s * RPT2 + g * 64
            for k in range(4):
                idxo[pl.ds(k * 16, 16)] = iota16 + (base + k * 16)
            pltpu.async_copy(st_sh.at[idxo], gxz0.at[pl.ds(0, 64)], gs0).wait()
            pltpu.sync_copy(gxz0.at[pl.ds(0, 64)],
                            st_hbm.at[c, p, pl.ds(base, 64)])
            pltpu.async_copy(u_sh.at[idxo], v3.at[pl.ds(0, 64)], gs0).wait()
            def _pack(i, _):
                for k in range(8):
                    gxz1[i, pl.ds(k * 16, 16)] = v3[2 * i + k // 4,
                                                     pl.ds((k % 4) * 16, 16)]
                return 0
            lax.fori_loop(0, 32, _pack, 0)
            pltpu.sync_copy(gxz1.at[pl.ds(0, 32)],
                            u_hbm.at[c, p, s, pl.ds(g * 32, 32)])
        plsc.subcore_barrier()


def _k4(src4b, dst4, ee4, XZcat):
    mesh = plsc.VectorSubcoreMesh(core_axis_name="c", subcore_axis_name="s",
                                  num_cores=NC, num_subcores=NS)
    f = pl.kernel(
        _k4_body,
        out_type=(jax.ShapeDtypeStruct((NC, 2, NPP, 128), _f32),
                  jax.ShapeDtypeStruct((NC, 2, NS, RPT2 // 2, 128), _f32)),
        mesh=mesh,
        compiler_params=pltpu.CompilerParams(needs_layout_passes=False),
        scratch_types=[
            pltpu.VMEM((GRP, SUB), _i32),
            pltpu.VMEM((GRP, SUB), _i32),
            pltpu.VMEM((GRP, SUB), _f32),
            pltpu.VMEM((SUB, 128), _f32),
            pltpu.VMEM((SUB, 128), _f32),
            pltpu.VMEM((SUB, H), _f32),
            pltpu.VMEM((SUB,), _i32),
            pltpu.VMEM((SUB,), _i32),
            pltpu.VMEM((112,), _i32),
            pltpu.VMEM((64,), _i32),
            pltpu.SemaphoreType.DMA,
            pltpu.SemaphoreType.DMA,
            pltpu.SemaphoreType.DMA,
            pltpu.SemaphoreType.DMA,
            pltpu.SemaphoreType.DMA,
            pltpu.SemaphoreType.DMA,
            pltpu.VMEM_SHARED((R2, 128), _f32),
            pltpu.VMEM_SHARED((R2, H), _f32),
        ],
    )
    return f(src4b, dst4, ee4, XZcat)


def _k5_body(x_ref, s_ref, t_ref, u_ref, np_ref,
             wgcn, bgcn, wsl, bsl, wsr, wg1, bg1, wg2, bg2, wgat, bgat,
             out_ref):
    x = x_ref[...]
    S = s_ref[...]
    T = t_ref[...]
    U = u_ref[...]
    npb = np_ref[...]
    dinv = npb[:, 0:1]
    sage = npb[:, 1:2]
    invden = npb[:, 2:3]
    ees = npb[:, 3:4]

    dot = functools.partial(jnp.dot, preferred_element_type=_f32)
    x1 = dot(dinv * T + (dinv * dinv) * x, wgcn[...]) + bgcn[...]
    x2 = dot(sage * S, wsl[...]) + bsl[...] + dot(x, wsr[...])
    hg = jnp.maximum(dot(x + S, wg1[...]) + bg1[...], 0.0)
    x3 = dot(hg, wg2[...]) + bg2[...]
    x4 = dot((U + ees * x) * invden, wgat[...]) + bgat[...]
    out_ref[...] = jnp.maximum(x1 + x2 + x3 + x4, 0.0)


def _k5(x, S, T, U, nodep, W_gcn, b_gcn, W_sage_l, b_sage_l, W_sage_r,
        W_gin1, b_gin1, W_gin2, b_gin2, W_gat, b_gat):
    full = lambda shape: pl.BlockSpec(shape, lambda i: tuple(0 for _ in shape))
    row = pl.BlockSpec((BLK, D), lambda i: (i, 0))
    return pl.pallas_call(
        _k5_body,
        grid=(N // BLK,),
        in_specs=[
            row, row, row, row,
            pl.BlockSpec((BLK, 8), lambda i: (i, 0)),
            full((D, D)), full((1, D)),
            full((D, D)), full((1, D)), full((D, D)),
            full((D, D)), full((1, D)), full((D, D)), full((1, D)),
            full((D, D)), full((1, D)),
        ],
        out_specs=row,
        out_shape=jax.ShapeDtypeStruct((N, D), _f32),
    )(x, S, T, U, nodep,
      W_gcn, b_gcn, W_sage_l, b_sage_l, W_sage_r,
      W_gin1, b_gin1, W_gin2, b_gin2, W_gat, b_gat)


def kernel(x, edge_index, W_gcn, b_gcn, W_sage_l, b_sage_l, W_sage_r,
           W_gin1, b_gin1, W_gin2, b_gin2, W_gat, a_src, a_dst, b_gat):
    src = edge_index[0]
    dst = edge_index[1]
    pad = E2 - E
    src_p = jnp.concatenate([src, jnp.zeros((pad,), _i32)])
    dst_p = jnp.concatenate([dst, jnp.full((pad,), N, _i32)])

    a2 = jnp.stack([a_src, a_dst], axis=1)
    al2, m2 = _k1(x, W_gat, a2)
    als = al2[:, 0] + 0.0
    ald = al2[:, 1] + 0.0

    src2 = src_p.reshape(NW, SUB2, SUB)
    dst2 = dst_p.reshape(NW, SUB2, SUB)
    ee, acc4, eeself = _k2(src2, dst2, als, ald, m2)

    acc = acc4.reshape(NC, R, 16)
    ees_col = eeself.reshape(R, 1)[:N]
    z, nodep = _k3(x, acc, ees_col)

    XZcat = jnp.concatenate(
        [jnp.concatenate([x[:, :H], z[:, :H]], axis=1),
         jnp.concatenate([x[:, H:], z[:, H:]], axis=1)], axis=0)
    src4 = src_p.reshape(NS, SUB4, SUB)
    src4b = jnp.stack([src4, src4 + N], axis=0)
    dst4 = dst_p.reshape(NS, SUB4, SUB)
    ee4 = ee.reshape(NS, SUB4, SUB)

    st4, u4 = _k4(src4b, dst4, ee4, XZcat)
    st = st4.reshape(NC, R, 128)
    S = jnp.concatenate([st[0, :N, :H], st[1, :N, :H]], axis=1)
    T = jnp.concatenate([st[0, :N, H:], st[1, :N, H:]], axis=1)
    u_r = u4.reshape(NC, R, H)
    U = jnp.concatenate([u_r[0, :N], u_r[1, :N]], axis=1)

    r2 = lambda b: b.reshape(1, D)
    return _k5(x, S, T, U, nodep, W_gcn, r2(b_gcn), W_sage_l, r2(b_sage_l),
               W_sage_r, W_gin1, r2(b_gin1), W_gin2, r2(b_gin2), W_gat,
               r2(b_gat))
